# Initial kernel scaffold; baseline (speedup 1.0000x reference)
import jax
import jax.numpy as jnp
from jax.experimental import pallas as pl


def kernel(m_ji, e_rbf, a_sbf, kj_idx, ji_idx, W_mkj, b_mkj, We1, We2, Wa1, Wa2, W_down, W_up):
    raise NotImplementedError("write your pallas kernel here")


# speedup vs baseline: 4.0780x; 4.0780x over previous
import functools

import jax
import jax.numpy as jnp
from jax import lax
from jax.experimental import pallas as pl
from jax.experimental.pallas import tpu as pltpu
from jax.experimental.pallas import tpu_sc as plsc

E = 320000
A = 640000
D = 128
INT = 64


_BE = 2560
_BA = 2560


def _k1_body(m_ref, e_ref, wm_ref, b_ref, we1_ref, we2_ref, mt_ref, et_ref):
    z = jnp.dot(m_ref[...], wm_ref[...], preferred_element_type=jnp.float32)
    z = z + b_ref[...]
    mt_ref[...] = z * jax.nn.sigmoid(z)
    e1 = jnp.dot(e_ref[...], we1_ref[...], preferred_element_type=jnp.float32)
    et_ref[...] = jnp.dot(e1, we2_ref[...], preferred_element_type=jnp.float32)


def _k3_body(h_ref, a_ref, wa1_ref, wa2_ref, wd_ref, em_ref):
    s = jnp.dot(h_ref[...], wd_ref[...], preferred_element_type=jnp.float32)
    s = s * jax.nn.sigmoid(s)
    a1 = jnp.dot(a_ref[...], wa1_ref[...], preferred_element_type=jnp.float32)
    ap = jnp.dot(a1, wa2_ref[...], preferred_element_type=jnp.float32)
    em_ref[...] = jnp.concatenate([s * ap, jnp.zeros_like(s)], axis=1)


def _k5_body(g_ref, wu_ref, o_ref):
    z = jnp.dot(g_ref[...], wu_ref[...], preferred_element_type=jnp.float32)
    o_ref[...] = z * jax.nn.sigmoid(z)


def _row_blocked(nrows, block, ncols):
    return pl.BlockSpec((block, ncols), lambda i: (i, 0))


def _full(shape):
    return pl.BlockSpec(shape, lambda i: tuple(0 for _ in shape))


_NW = 32
_B2 = 256
_NBLK2 = A // _B2


def _k2_body(mt_hbm, et_hbm, kj2_hbm, ji2_hbm, h_hbm,
             kjb, jib, mrows, erows, sem):
    c = lax.axis_index("c")
    s = lax.axis_index("s")
    wid = s * 2 + c
    nfull = _NBLK2 // _NW
    rem = _NBLK2 % _NW
    trips = jnp.where(wid < rem, nfull + 1, nfull)

    @pl.loop(0, trips)
    def _block(g):
        blk = wid + g * _NW
        base = blk * _B2
        pltpu.sync_copy(kj2_hbm.at[pl.ds(blk * 2, 2)], kjb)
        pltpu.sync_copy(ji2_hbm.at[pl.ds(blk * 2, 2)], jib)
        descs = []
        for j in range(2):
            descs.append(pltpu.async_copy(
                mt_hbm.at[kjb.at[j]], mrows.at[pl.ds(j * 128, 128)], sem))
            descs.append(pltpu.async_copy(
                et_hbm.at[jib.at[j]], erows.at[pl.ds(j * 128, 128)], sem))
        for d in descs:
            d.wait()

        @pl.loop(0, _B2)
        def _row(r):
            for cc in range(D // 16):
                sl = pl.ds(cc * 16, 16)
                mrows[r, sl] = mrows[r, sl] * erows[r, sl]

        pltpu.sync_copy(mrows, h_hbm.at[pl.ds(base, _B2)])


def _k2_call(mt, et, kj2, ji2):
    mesh = plsc.VectorSubcoreMesh(core_axis_name="c", subcore_axis_name="s")
    f = pl.kernel(
        _k2_body,
        out_type=jax.ShapeDtypeStruct((A, D), jnp.float32),
        mesh=mesh,
        compiler_params=pltpu.CompilerParams(needs_layout_passes=False),
        scratch_types=[
            pltpu.VMEM((2, 128), jnp.int32),
            pltpu.VMEM((2, 128), jnp.int32),
            pltpu.VMEM((_B2, D), jnp.float32),
            pltpu.VMEM((_B2, D), jnp.float32),
            pltpu.SemaphoreType.DMA,
        ],
    )
    return f(mt, et, kj2, ji2)


_B4 = 512
_NBLK4 = A // _B4
_RSC = 13568
_TRASH = 128
_SROWS = _RSC + _TRASH
_NPASS = -(-E // (2 * _RSC))
_RING = 256


def _k4_flush(start, em_hbm, emb, posf, dstf, dst2, shared):
    for cc in range(8):
        sl = pl.ds(cc * 16, 16)
        dst2[0, sl] = posf[pl.ds(start + cc * 16, 16)]
        dst2[1, sl] = dstf[pl.ds(start + cc * 16, 16)]
    pltpu.sync_copy(em_hbm.at[dst2.at[0]], emb)
    pltpu.sync_copy(emb, shared.at[dst2.at[1]], add=True)


def _k4_body(ji2_hbm, em_hbm, agg_hbm, jib, emb, posf, dstf, dst2, shared):
    c = lax.axis_index("c")
    s = lax.axis_index("s")
    nfull = _NBLK4 // 16
    rem = _NBLK4 % 16
    trips = jnp.where(s < rem, nfull + 1, nfull)
    iota = lax.broadcasted_iota(jnp.int32, (16,), 0)
    z16 = jnp.zeros((16,), jnp.float32)
    zi16 = jnp.zeros((16,), jnp.int32)

    @pl.loop(0, _NPASS)
    def _pass(p):
        lo = p * 2 * _RSC + c * _RSC

        @pl.loop(0, 128)
        def _zr(r):
            for cc in range(D // 16):
                emb[r, pl.ds(cc * 16, 16)] = z16

        share = _SROWS // 16
        sbase = s * share
        for k in range(share // 128):
            pltpu.sync_copy(emb, shared.at[pl.ds(sbase + k * 128, 128)])
        tail = share % 128
        if tail:
            pltpu.sync_copy(emb.at[pl.ds(0, tail)],
                            shared.at[pl.ds(sbase + (share // 128) * 128, tail)])
        plsc.subcore_barrier()

        def _vreg(off, done, idxv, posg):
            d = idxv - lo
            m = (d >= 0) & (d < _RSC)
            mi = m.astype(jnp.int32)
            slot = (off + plsc.cumsum(mi) - 1) & (_RING - 1)
            plsc.store_scatter(posf, [slot], posg, mask=m)
            plsc.store_scatter(dstf, [slot], d, mask=m)
            off = off + jnp.sum(mi)
            full = off - done >= 128

            @pl.when(full)
            def _():
                _k4_flush(done & (_RING - 1), em_hbm, emb, posf, dstf, dst2,
                          shared)

            return off, jnp.where(full, done + 128, done)

        @pl.loop(0, trips, init_carry=(jnp.int32(0), jnp.int32(0)))
        def _blk(g, carry):
            off, done = carry
            blk = s + g * 16
            pltpu.sync_copy(ji2_hbm.at[pl.ds(blk * 4, 4)], jib)
            base = blk * _B4
            for j in range(4):
                for cc in range(8):
                    idxv = jib[j, pl.ds(cc * 16, 16)]
                    posg = (base + j * 128 + cc * 16) + iota
                    off, done = _vreg(off, done, idxv, posg)
            return off, done

        off, done = _blk
        for k in range(8):
            slot = (off + k * 16 + iota) & (_RING - 1)
            plsc.store_scatter(posf, [slot], zi16, mask=None)
            plsc.store_scatter(dstf, [slot], _RSC + k * 16 + iota, mask=None)

        @pl.when(off - done > 0)
        def _():
            _k4_flush(done & (_RING - 1), em_hbm, emb, posf, dstf, dst2,
                      shared)

        plsc.subcore_barrier()
        dshare = _RSC // 16
        pltpu.sync_copy(shared.at[pl.ds(s * dshare, dshare)],
                        agg_hbm.at[pl.ds(lo + s * dshare, dshare)])
        plsc.subcore_barrier()


def _k4_call(ji2, em):
    mesh = plsc.VectorSubcoreMesh(core_axis_name="c", subcore_axis_name="s")
    f = pl.kernel(
        _k4_body,
        out_type=jax.ShapeDtypeStruct((_NPASS * 2 * _RSC, D), jnp.float32),
        mesh=mesh,
        compiler_params=pltpu.CompilerParams(needs_layout_passes=False),
        scratch_types=[
            pltpu.VMEM((4, 128), jnp.int32),
            pltpu.VMEM((128, D), jnp.float32),
            pltpu.VMEM((_RING,), jnp.int32),
            pltpu.VMEM((_RING,), jnp.int32),
            pltpu.VMEM((2, 128), jnp.int32),
            pltpu.VMEM_SHARED((_SROWS, D), jnp.float32),
        ],
    )
    return f(ji2, em)


@jax.jit
def kernel(m_ji, e_rbf, a_sbf, kj_idx, ji_idx,
           W_mkj, b_mkj, We1, We2, Wa1, Wa2, W_down, W_up):
    b2 = b_mkj.reshape(1, D)
    mt, et = pl.pallas_call(
        _k1_body,
        grid=(E // _BE,),
        in_specs=[_row_blocked(E, _BE, D), _row_blocked(E, _BE, 6),
                  _full((D, D)), _full((1, D)), _full((6, 8)), _full((8, D))],
        out_specs=[_row_blocked(E, _BE, D), _row_blocked(E, _BE, D)],
        out_shape=[jax.ShapeDtypeStruct((E, D), jnp.float32),
                   jax.ShapeDtypeStruct((E, D), jnp.float32)],
    )(m_ji, e_rbf, W_mkj, b2, We1, We2)

    kj2 = kj_idx.astype(jnp.int32).reshape(A // 128, 128)
    ji2 = ji_idx.astype(jnp.int32).reshape(A // 128, 128)

    h = _k2_call(mt, et, kj2, ji2)

    em = pl.pallas_call(
        _k3_body,
        grid=(A // _BA,),
        in_specs=[_row_blocked(A, _BA, D), _row_blocked(A, _BA, 42),
                  _full((42, 8)), _full((8, INT)), _full((D, INT))],
        out_specs=_row_blocked(A, _BA, D),
        out_shape=jax.ShapeDtypeStruct((A, D), jnp.float32),
    )(h, a_sbf, Wa1, Wa2, W_down)

    agg = _k4_call(ji2, em)

    W_up_p = jnp.zeros((D, D), jnp.float32).at[:INT].set(W_up)
    out = pl.pallas_call(
        _k5_body,
        grid=(E // _BE,),
        in_specs=[_row_blocked(E, _BE, D), _full((D, D))],
        out_specs=_row_blocked(E, _BE, D),
        out_shape=jax.ShapeDtypeStruct((E, D), jnp.float32),
    )(agg, W_up_p)
    return out


# speedup vs baseline: 6.6349x; 1.6270x over previous
import functools

import jax
import jax.numpy as jnp
from jax import lax
from jax.experimental import pallas as pl
from jax.experimental.pallas import tpu as pltpu
from jax.experimental.pallas import tpu_sc as plsc

E = 320000
A = 640000
D = 128
INT = 64


_BE = 2560
_BA = 2560


def _k1_body(m_ref, e_ref, wm_ref, b_ref, we1_ref, we2_ref, mt_ref, et_ref):
    z = jnp.dot(m_ref[...], wm_ref[...], preferred_element_type=jnp.float32)
    z = z + b_ref[...]
    mt_ref[...] = z * jax.nn.sigmoid(z)
    e1 = jnp.dot(e_ref[...], we1_ref[...], preferred_element_type=jnp.float32)
    et_ref[...] = jnp.dot(e1, we2_ref[...], preferred_element_type=jnp.float32)


def _k3_body(h_ref, a_ref, wa1_ref, wa2_ref, wd_ref, em_ref):
    s = jnp.dot(h_ref[...], wd_ref[...], preferred_element_type=jnp.float32)
    s = s * jax.nn.sigmoid(s)
    a1 = jnp.dot(a_ref[...], wa1_ref[...], preferred_element_type=jnp.float32)
    ap = jnp.dot(a1, wa2_ref[...], preferred_element_type=jnp.float32)
    em_ref[...] = jnp.concatenate([s * ap, jnp.zeros_like(s)], axis=1)


def _k5_body(g_ref, wu_ref, o_ref):
    z = jnp.dot(g_ref[...], wu_ref[...], preferred_element_type=jnp.float32)
    o_ref[...] = z * jax.nn.sigmoid(z)


def _row_blocked(nrows, block, ncols):
    return pl.BlockSpec((block, ncols), lambda i: (i, 0))


def _full(shape):
    return pl.BlockSpec(shape, lambda i: tuple(0 for _ in shape))


_NW = 32
_B2 = 256
_NBLK2 = A // _B2


def _k2_body(mt_hbm, et_hbm, kj2_hbm, ji2_hbm, h_hbm,
             kjb, jib, mrows, erows, sem):
    c = lax.axis_index("c")
    s = lax.axis_index("s")
    wid = s * 2 + c
    nfull = _NBLK2 // _NW
    rem = _NBLK2 % _NW
    trips = jnp.where(wid < rem, nfull + 1, nfull)

    @pl.loop(0, trips)
    def _block(g):
        blk = wid + g * _NW
        base = blk * _B2
        pltpu.sync_copy(kj2_hbm.at[pl.ds(blk * 2, 2)], kjb)
        pltpu.sync_copy(ji2_hbm.at[pl.ds(blk * 2, 2)], jib)
        descs = []
        for j in range(2):
            descs.append(pltpu.async_copy(
                mt_hbm.at[kjb.at[j]], mrows.at[pl.ds(j * 128, 128)], sem))
            descs.append(pltpu.async_copy(
                et_hbm.at[jib.at[j]], erows.at[pl.ds(j * 128, 128)], sem))
        for d in descs:
            d.wait()

        @pl.loop(0, _B2)
        def _row(r):
            for cc in range(D // 16):
                sl = pl.ds(cc * 16, 16)
                mrows[r, sl] = mrows[r, sl] * erows[r, sl]

        pltpu.sync_copy(mrows, h_hbm.at[pl.ds(base, _B2)])


def _k2_call(mt, et, kj2, ji2):
    mesh = plsc.VectorSubcoreMesh(core_axis_name="c", subcore_axis_name="s")
    f = pl.kernel(
        _k2_body,
        out_type=jax.ShapeDtypeStruct((A, D), jnp.float32),
        mesh=mesh,
        compiler_params=pltpu.CompilerParams(needs_layout_passes=False),
        scratch_types=[
            pltpu.VMEM((2, 128), jnp.int32),
            pltpu.VMEM((2, 128), jnp.int32),
            pltpu.VMEM((_B2, D), jnp.float32),
            pltpu.VMEM((_B2, D), jnp.float32),
            pltpu.SemaphoreType.DMA,
        ],
    )
    return f(mt, et, kj2, ji2)


_B4 = 512
_NBLK4 = A // _B4
_RSC = 13568
_TRASH = 128
_SROWS = _RSC + _TRASH
_NPASS = -(-E // (2 * _RSC))
_RING = 256
_FB = 64


def _k4_body(ji2_hbm, em_hbm, agg_hbm, jib, emb, posf, dstf, snap, shared,
             sem_i, sem_g):
    c = lax.axis_index("c")
    s = lax.axis_index("s")
    nfull = _NBLK4 // 16
    rem = _NBLK4 % 16
    trips = jnp.where(s < rem, nfull + 1, nfull)
    iota = lax.broadcasted_iota(jnp.int32, (16,), 0)
    z16 = jnp.zeros((16,), jnp.float32)
    zi16 = jnp.zeros((16,), jnp.int32)

    def _wait_gather():
        pltpu.make_async_copy(em_hbm.at[snap.at[0, 0]],
                              emb.at[pl.ds(0, _FB)], sem_g).wait()

    def _flush(done):
        fpar = (done // _FB) & 1
        ppar = 1 - fpar
        _wait_gather()
        pltpu.sync_copy(emb.at[pl.ds(ppar * _FB, _FB)],
                        shared.at[snap.at[ppar, 1]], add=True)
        start = done & (_RING - 1)
        for cc in range(_FB // 16):
            sl = pl.ds(cc * 16, 16)
            snap[fpar, 0, sl] = posf[pl.ds(start + cc * 16, 16)]
            snap[fpar, 1, sl] = dstf[pl.ds(start + cc * 16, 16)]
        pltpu.async_copy(em_hbm.at[snap.at[fpar, 0]],
                         emb.at[pl.ds(fpar * _FB, _FB)], sem_g)

    @pl.loop(0, _NPASS)
    def _pass(p):
        lo = p * 2 * _RSC + c * _RSC

        @pl.loop(0, 128)
        def _zr(r):
            for cc in range(D // 16):
                emb[r, pl.ds(cc * 16, 16)] = z16

        share = _SROWS // 16
        sbase = s * share
        for k in range(share // 128):
            pltpu.sync_copy(emb, shared.at[pl.ds(sbase + k * 128, 128)])
        tail = share % 128
        if tail:
            pltpu.sync_copy(emb.at[pl.ds(0, tail)],
                            shared.at[pl.ds(sbase + (share // 128) * 128, tail)])
        plsc.subcore_barrier()

        for par in range(2):
            for cc in range(_FB // 16):
                sl = pl.ds(cc * 16, 16)
                snap[par, 0, sl] = zi16
                snap[par, 1, sl] = _RSC + cc * 16 + iota
        pltpu.async_copy(em_hbm.at[snap.at[1, 0]],
                         emb.at[pl.ds(_FB, _FB)], sem_g)
        pltpu.async_copy(ji2_hbm.at[pl.ds(s * 4, 4)], jib.at[0], sem_i)
        pltpu.async_copy(ji2_hbm.at[pl.ds((s + 16) * 4, 4)], jib.at[1], sem_i)

        @pl.loop(0, trips, init_carry=(jnp.int32(0), jnp.int32(0)))
        def _blk(g, carry):
            off, done = carry
            blk = s + g * 16
            gpar = g & 1
            pltpu.make_async_copy(ji2_hbm.at[pl.ds(blk * 4, 4)],
                                  jib.at[gpar], sem_i).wait()
            base = blk * _B4
            for j in range(4):
                for cc in range(8):
                    idxv = jib[gpar, j, pl.ds(cc * 16, 16)]
                    d = idxv - lo
                    m = (d >= 0) & (d < _RSC)
                    mi = m.astype(jnp.int32)
                    slot = (off + plsc.cumsum(mi) - 1) & (_RING - 1)
                    posg = (base + j * 128 + cc * 16) + iota
                    plsc.store_scatter(posf, [slot], posg, mask=m)
                    plsc.store_scatter(dstf, [slot], d, mask=m)
                    off = off + jnp.sum(mi)
                    if cc % 4 == 3:
                        full = off - done >= _FB

                        @pl.when(full)
                        def _():
                            _flush(done)

                        done = jnp.where(full, done + _FB, done)

            @pl.when(g + 2 < trips)
            def _():
                blk2 = s + (g + 2) * 16
                pltpu.async_copy(ji2_hbm.at[pl.ds(blk2 * 4, 4)],
                                 jib.at[gpar], sem_i)

            return off, done

        off, done = _blk
        for k in range(_FB // 16):
            slot = (off + k * 16 + iota) & (_RING - 1)
            plsc.store_scatter(posf, [slot], zi16, mask=None)
            plsc.store_scatter(dstf, [slot], _RSC + k * 16 + iota, mask=None)
        _flush(done)
        done = done + _FB
        ppar2 = 1 - ((done // _FB) & 1)
        _wait_gather()
        pltpu.sync_copy(emb.at[pl.ds(ppar2 * _FB, _FB)],
                        shared.at[snap.at[ppar2, 1]], add=True)

        plsc.subcore_barrier()
        dshare = _RSC // 16
        pltpu.sync_copy(shared.at[pl.ds(s * dshare, dshare)],
                        agg_hbm.at[pl.ds(lo + s * dshare, dshare)])
        plsc.subcore_barrier()


def _k4_call(ji2, em):
    mesh = plsc.VectorSubcoreMesh(core_axis_name="c", subcore_axis_name="s")
    f = pl.kernel(
        _k4_body,
        out_type=jax.ShapeDtypeStruct((_NPASS * 2 * _RSC, D), jnp.float32),
        mesh=mesh,
        compiler_params=pltpu.CompilerParams(needs_layout_passes=False),
        scratch_types=[
            pltpu.VMEM((2, 4, 128), jnp.int32),
            pltpu.VMEM((2 * _FB, D), jnp.float32),
            pltpu.VMEM((_RING,), jnp.int32),
            pltpu.VMEM((_RING,), jnp.int32),
            pltpu.VMEM((2, 2, _FB), jnp.int32),
            pltpu.VMEM_SHARED((_SROWS, D), jnp.float32),
            pltpu.SemaphoreType.DMA,
            pltpu.SemaphoreType.DMA,
        ],
    )
    return f(ji2, em)


@jax.jit
def kernel(m_ji, e_rbf, a_sbf, kj_idx, ji_idx,
           W_mkj, b_mkj, We1, We2, Wa1, Wa2, W_down, W_up):
    b2 = b_mkj.reshape(1, D)
    mt, et = pl.pallas_call(
        _k1_body,
        grid=(E // _BE,),
        in_specs=[_row_blocked(E, _BE, D), _row_blocked(E, _BE, 6),
                  _full((D, D)), _full((1, D)), _full((6, 8)), _full((8, D))],
        out_specs=[_row_blocked(E, _BE, D), _row_blocked(E, _BE, D)],
        out_shape=[jax.ShapeDtypeStruct((E, D), jnp.float32),
                   jax.ShapeDtypeStruct((E, D), jnp.float32)],
    )(m_ji, e_rbf, W_mkj, b2, We1, We2)

    kj2 = kj_idx.astype(jnp.int32).reshape(A // 128, 128)
    ji2 = ji_idx.astype(jnp.int32).reshape(A // 128, 128)

    h = _k2_call(mt, et, kj2, ji2)

    em = pl.pallas_call(
        _k3_body,
        grid=(A // _BA,),
        in_specs=[_row_blocked(A, _BA, D), _row_blocked(A, _BA, 42),
                  _full((42, 8)), _full((8, INT)), _full((D, INT))],
        out_specs=_row_blocked(A, _BA, D),
        out_shape=jax.ShapeDtypeStruct((A, D), jnp.float32),
    )(h, a_sbf, Wa1, Wa2, W_down)

    agg = _k4_call(ji2, em)

    W_up_p = jnp.zeros((D, D), jnp.float32).at[:INT].set(W_up)
    out = pl.pallas_call(
        _k5_body,
        grid=(E // _BE,),
        in_specs=[_row_blocked(E, _BE, D), _full((D, D))],
        out_specs=_row_blocked(E, _BE, D),
        out_shape=jax.ShapeDtypeStruct((E, D), jnp.float32),
    )(agg, W_up_p)
    return out
